# baseline (speedup 1.0000x reference)
import jax
import jax.numpy as jnp
from jax import lax
from jax.experimental import pallas as pl
from jax.experimental.pallas import tpu as pltpu
from jax.experimental.pallas import tpu_sc as plsc

NC = 2
NS = 16
L = 16


def _lut_kernel(n_rows, w, n_b, n_c, rows_per_w, row_chunk):
    n_chunks = rows_per_w // row_chunk
    cols = w // L
    hp_rows = 256
    hp_per_w = rows_per_w // hp_rows
    chunks_per_hp = hp_rows // row_chunk
    mesh = plsc.VectorSubcoreMesh(
        core_axis_name="c", subcore_axis_name="s",
        num_cores=NC, num_subcores=NS)

    def body(im_hbm, it_hbm, out_hbm, lut_v,
             in0, in1, ou0, ou1, sin0, sin1, sou0, sou1):
        cid = lax.axis_index("c")
        sid = lax.axis_index("s")
        wid = sid * NC + cid
        base = wid * rows_per_w
        ins, ous = (in0, in1), (ou0, ou1)
        sins, sous = (sin0, sin1), (sou0, sou1)

        def in_slice(c):
            return im_hbm.at[pl.ds(base + c * row_chunk, row_chunk)]

        def out_slice(c):
            return out_hbm.at[pl.ds(base + c * row_chunk, row_chunk)]

        pltpu.async_copy(in_slice(0), in0, sin0)
        pltpu.async_copy(in_slice(1), in1, sin1)

        @pl.loop(0, n_chunks, step=2)
        def _c2(c0):
            for par in range(2):
                c = c0 + par
                inb, oub, sin, sou = ins[par], ous[par], sins[par], sous[par]

                @pl.when(c % chunks_per_hp == 0)
                def _():
                    hp = wid * hp_per_w + c // chunks_per_hp
                    p = hp // 2
                    pltpu.sync_copy(
                        it_hbm.at[(p % n_c) * n_b + p // n_c], lut_v)

                pltpu.make_async_copy(in_slice(c), inb, sin).wait()
                @pl.when(c0 >= 2)
                def _():
                    pltpu.make_async_copy(oub, out_slice(c), sou).wait()

                @plsc.parallel_loop(0, row_chunk, unroll=2)
                def _row(r):
                    for t in range(cols):
                        s = t * L
                        x = inb[r, pl.ds(s, L)]
                        idx = (x * 255.0 + 0.5).astype(jnp.int32)
                        oub[r, pl.ds(s, L)] = plsc.load_gather(lut_v, [idx])

                pltpu.async_copy(oub, out_slice(c), sou)

                @pl.when(c0 + 2 < n_chunks)
                def _():
                    pltpu.async_copy(in_slice(c + 2), inb, sin)

        pltpu.make_async_copy(ou0, out_slice(n_chunks - 2), sou0).wait()
        pltpu.make_async_copy(ou1, out_slice(n_chunks - 1), sou1).wait()

    return pl.kernel(
        body,
        out_type=jax.ShapeDtypeStruct((n_rows, w), jnp.float32),
        mesh=mesh,
        scratch_types=[
            pltpu.VMEM((256,), jnp.float32),
            pltpu.VMEM((row_chunk, w), jnp.float32),
            pltpu.VMEM((row_chunk, w), jnp.float32),
            pltpu.VMEM((row_chunk, w), jnp.float32),
            pltpu.VMEM((row_chunk, w), jnp.float32),
            pltpu.SemaphoreType.DMA,
            pltpu.SemaphoreType.DMA,
            pltpu.SemaphoreType.DMA,
            pltpu.SemaphoreType.DMA,
        ],
        compiler_params=pltpu.CompilerParams(needs_layout_passes=False),
    )


def kernel(im, it):
    B, H, W, C = im.shape
    n_rows = B * C * H
    nw = NC * NS
    rows_per_w = n_rows // nw
    row_chunk = 32
    assert rows_per_w % row_chunk == 0
    im2 = im.transpose(0, 3, 1, 2).reshape(n_rows, W)
    it2 = it.transpose(2, 0, 1).reshape(C * B, it.shape[1])
    k = _lut_kernel(n_rows, W, B, C, rows_per_w, row_chunk)
    out = k(im2, it2)
    return out.reshape(B, C, H, W).transpose(0, 2, 3, 1)


# speedup vs baseline: 1.3536x; 1.3536x over previous
import jax
import jax.numpy as jnp
from jax import lax
from jax.experimental import pallas as pl
from jax.experimental.pallas import tpu as pltpu
from jax.experimental.pallas import tpu_sc as plsc

NC = 2
NS = 16
L = 16


def _lut_kernel(n_rows, w, n_b, n_c, rows_per_w, row_chunk):
    n_chunks = rows_per_w // row_chunk
    cols = w // L
    hp_rows = 256
    hp_per_w = rows_per_w // hp_rows
    chunks_per_hp = hp_rows // row_chunk
    mesh = plsc.VectorSubcoreMesh(
        core_axis_name="c", subcore_axis_name="s",
        num_cores=NC, num_subcores=NS)

    def body(im_hbm, it_hbm, out_hbm, lut_v,
             in0, in1, ou0, ou1, sin0, sin1, sou0, sou1):
        cid = lax.axis_index("c")
        sid = lax.axis_index("s")
        wid = sid * NC + cid
        base = wid * rows_per_w
        ins, ous = (in0, in1), (ou0, ou1)
        sins, sous = (sin0, sin1), (sou0, sou1)

        def in_slice(c):
            return im_hbm.at[pl.ds(base + c * row_chunk, row_chunk)]

        def out_slice(c):
            return out_hbm.at[pl.ds(base + c * row_chunk, row_chunk)]

        pltpu.async_copy(in_slice(0), in0, sin0)
        pltpu.async_copy(in_slice(1), in1, sin1)

        @pl.loop(0, n_chunks, step=2)
        def _c2(c0):
            for par in range(2):
                c = c0 + par
                inb, oub, sin, sou = ins[par], ous[par], sins[par], sous[par]

                @pl.when(c % chunks_per_hp == 0)
                def _():
                    hp = wid * hp_per_w + c // chunks_per_hp
                    p = hp // 2
                    pltpu.sync_copy(
                        it_hbm.at[(p % n_c) * n_b + p // n_c], lut_v)

                pltpu.make_async_copy(in_slice(c), inb, sin).wait()
                @pl.when(c0 >= 2)
                def _():
                    pltpu.make_async_copy(oub, out_slice(c), sou).wait()

                @plsc.parallel_loop(0, row_chunk)
                def _row(r):
                    for t in range(cols):
                        s = t * L
                        x = inb[r, pl.ds(s, L)]
                        t23 = x * 255.0 + 8388608.0
                        idx = plsc.bitcast(t23, jnp.int32) & 255
                        oub[r, pl.ds(s, L)] = plsc.load_gather(lut_v, [idx])

                pltpu.async_copy(oub, out_slice(c), sou)

                @pl.when(c0 + 2 < n_chunks)
                def _():
                    pltpu.async_copy(in_slice(c + 2), inb, sin)

        pltpu.make_async_copy(ou0, out_slice(n_chunks - 2), sou0).wait()
        pltpu.make_async_copy(ou1, out_slice(n_chunks - 1), sou1).wait()

    return pl.kernel(
        body,
        out_type=jax.ShapeDtypeStruct((n_rows, w), jnp.float32),
        mesh=mesh,
        scratch_types=[
            pltpu.VMEM((256,), jnp.float32),
            pltpu.VMEM((row_chunk, w), jnp.float32),
            pltpu.VMEM((row_chunk, w), jnp.float32),
            pltpu.VMEM((row_chunk, w), jnp.float32),
            pltpu.VMEM((row_chunk, w), jnp.float32),
            pltpu.SemaphoreType.DMA,
            pltpu.SemaphoreType.DMA,
            pltpu.SemaphoreType.DMA,
            pltpu.SemaphoreType.DMA,
        ],
        compiler_params=pltpu.CompilerParams(needs_layout_passes=False),
    )


def kernel(im, it):
    B, H, W, C = im.shape
    n_rows = B * C * H
    nw = NC * NS
    rows_per_w = n_rows // nw
    row_chunk = 32
    assert rows_per_w % row_chunk == 0
    im2 = im.transpose(0, 3, 1, 2).reshape(n_rows, W)
    it2 = it.transpose(2, 0, 1).reshape(C * B, it.shape[1])
    k = _lut_kernel(n_rows, W, B, C, rows_per_w, row_chunk)
    out = k(im2, it2)
    return out.reshape(B, C, H, W).transpose(0, 2, 3, 1)
